# baseline (speedup 1.0000x reference)
import functools

import jax
import jax.numpy as jnp
from jax import lax
from jax.experimental import pallas as pl
from jax.experimental.pallas import tpu as pltpu
from jax.experimental.pallas import tpu_sc as plsc

DIM = 64
LANES = 16
CHUNK = 128
BLOCKS = CHUNK // LANES


def _sqrt(s):
    bits = lax.bitcast_convert_type(s, jnp.int32)
    y = lax.bitcast_convert_type(
        jnp.int32(0x5F3759DF) - lax.shift_right_logical(bits, 1), jnp.float32)
    for _ in range(3):
        y = y * (1.5 - 0.5 * s * y * y)
    return s * y


def _sc_kernel(batch, n_workers):
    rows_per_worker = batch // n_workers
    n_chunks = rows_per_worker // CHUNK
    mesh = plsc.VectorSubcoreMesh(core_axis_name="c", subcore_axis_name="s")

    @functools.partial(
        pl.kernel,
        mesh=mesh,
        compiler_params=pltpu.CompilerParams(
            needs_layout_passes=False, use_tc_tiling_on_sc=True),
        out_type=jax.ShapeDtypeStruct((batch,), jnp.float32),
        scratch_types=[
            pltpu.VMEM((CHUNK,), jnp.int32),
            pltpu.VMEM((CHUNK,), jnp.int32),
            pltpu.VMEM((CHUNK,), jnp.int32),
            pltpu.VMEM((CHUNK,), jnp.int32),
            pltpu.VMEM((CHUNK,), jnp.int32),
            pltpu.VMEM((CHUNK,), jnp.int32),
            pltpu.VMEM((CHUNK, 2 * DIM), jnp.float32),
            pltpu.VMEM((CHUNK, 2 * DIM), jnp.float32),
            pltpu.VMEM((CHUNK, 2 * DIM), jnp.float32),
            pltpu.VMEM((CHUNK,), jnp.float32),
            pltpu.SemaphoreType.DMA,
        ],
    )
    def k(heads, relations, tails, entity2, relation2, out,
          hidx, ridx, tidx, hq, rq, tq, hrows, rrows, trows, outc, sem):
        n_cores = 2
        wid = lax.axis_index("s") * n_cores + lax.axis_index("c")
        base = wid * rows_per_worker
        lane_iota = lax.iota(jnp.int32, LANES)

        def chunk_body(c, _):
            off = base + c * CHUNK
            pltpu.sync_copy(heads.at[pl.ds(off, CHUNK)], hidx)
            pltpu.sync_copy(relations.at[pl.ds(off, CHUNK)], ridx)
            pltpu.sync_copy(tails.at[pl.ds(off, CHUNK)], tidx)
            for i in range(BLOCKS):
                sl = pl.ds(i * LANES, LANES)
                hq[sl] = lax.shift_right_logical(hidx[sl], 1)
                rq[sl] = lax.shift_right_logical(ridx[sl], 1)
                tq[sl] = lax.shift_right_logical(tidx[sl], 1)
            ch = pltpu.async_copy(entity2.at[hq], hrows, sem)
            cr = pltpu.async_copy(relation2.at[rq], rrows, sem)
            ct = pltpu.async_copy(entity2.at[tq], trows, sem)
            ch.wait()
            cr.wait()
            ct.wait()

            def blk_body(b, _):
                sl = pl.ds(b * LANES, LANES)
                rows = b * LANES + lane_iota
                hcol = (hidx[sl] & 1) * DIM
                rcol = (ridx[sl] & 1) * DIM
                tcol = (tidx[sl] & 1) * DIM
                acc = jnp.zeros((LANES,), jnp.float32)
                for d in range(DIM):
                    xh = plsc.load_gather(hrows, [rows, hcol + d])
                    xr = plsc.load_gather(rrows, [rows, rcol + d])
                    xt = plsc.load_gather(trows, [rows, tcol + d])
                    dd = xh + xr - xt
                    acc = acc + dd * dd
                outc[sl] = _sqrt(acc)
                return 0

            lax.fori_loop(0, BLOCKS, blk_body, 0)
            pltpu.sync_copy(outc, out.at[pl.ds(off, CHUNK)])
            return 0

        lax.fori_loop(0, n_chunks, chunk_body, 0)

    return k


def kernel(heads, relations, tails, entity_emb, relation_emb):
    batch = heads.shape[0]
    n_ent, dim = entity_emb.shape
    entity2 = entity_emb.reshape(n_ent // 2, 2 * dim)
    relation2 = relation_emb.reshape(relation_emb.shape[0] // 2, 2 * dim)
    k = _sc_kernel(batch, 32)
    return k(heads.astype(jnp.int32), relations.astype(jnp.int32),
             tails.astype(jnp.int32), entity2, relation2)
